# baseline (speedup 1.0000x reference)
import functools

import jax
import jax.numpy as jnp
from jax import lax
from jax.experimental import pallas as pl
from jax.experimental.pallas import tpu as pltpu
from jax.experimental.pallas import tpu_sc as plsc

N = 10000
E = 320000
H = 128
HH = H // 2
L = 4
R = 3

NC = 2
NS = 16

STRIDE = 10000
ACC_ROWS = R * STRIDE
ROWCHUNK = 80
N_ROWCHUNKS = ACC_ROWS // ROWCHUNK
CHUNK = 64
EP_TILE = 20480
EP = NS * EP_TILE
N_CHUNKS = EP_TILE // CHUNK
BPB = 8
NB = N_CHUNKS // BPB
NXR = 8

BN = 80
GRID = N // BN


def _seg_kernel(x_hbm, idx_hbm, zeros_hbm, out_hbm,
                buf_a, buf_b, rows_a, rows_b, acc_sh,
                sem_ia, sem_ib, sem_a, sem_b):
    c = lax.axis_index("c")
    s = lax.axis_index("s")

    def fetch_idx(blk, buf, isem):
        pltpu.async_copy(idx_hbm.at[c, s, blk], buf, isem)

    def wait_idx(blk, buf, isem):
        pltpu.make_async_copy(idx_hbm.at[c, s, blk], buf, isem).wait()

    def issue_gather(buf, u, rows, sem):
        pltpu.async_copy(x_hbm.at[buf.at[u, 0]], rows, sem)

    def wait_gather(buf, u, rows, sem):
        pltpu.make_async_copy(x_hbm.at[buf.at[u, 0]], rows, sem).wait()

    def issue_scat(buf, u, rows, sem):
        pltpu.async_copy(rows, acc_sh.at[buf.at[u, 1]], sem, add=True)

    def wait_scat(buf, u, rows, sem):
        pltpu.make_async_copy(rows, acc_sh.at[buf.at[u, 1]], sem).wait()

    fetch_idx(0, buf_a, sem_ia)
    fetch_idx(1, buf_b, sem_ib)

    @pl.loop(0, pl.cdiv(N_ROWCHUNKS, NS))
    def _(i):
        k = i * NS + s

        @pl.when(k < N_ROWCHUNKS)
        def _():
            pltpu.sync_copy(zeros_hbm, acc_sh.at[pl.ds(k * ROWCHUNK, ROWCHUNK)])

    wait_idx(0, buf_a, sem_ia)
    issue_gather(buf_a, 0, rows_a, sem_a)
    issue_gather(buf_a, 1, rows_b, sem_b)
    plsc.subcore_barrier()

    def do_block(blk, buf, isem, obuf, oisem):
        for p in range(BPB // 2):
            u = 2 * p
            wait_gather(buf, u, rows_a, sem_a)
            issue_scat(buf, u, rows_a, sem_a)
            wait_gather(buf, u + 1, rows_b, sem_b)
            issue_scat(buf, u + 1, rows_b, sem_b)
            wait_scat(buf, u, rows_a, sem_a)
            if u + 2 < BPB:
                issue_gather(buf, u + 2, rows_a, sem_a)
            else:
                @pl.when(blk + 1 < NB)
                def _():
                    wait_idx(blk + 1, obuf, oisem)
                    issue_gather(obuf, 0, rows_a, sem_a)
            wait_scat(buf, u + 1, rows_b, sem_b)
            if u + 3 < BPB:
                issue_gather(buf, u + 3, rows_b, sem_b)
            else:
                @pl.when(blk + 1 < NB)
                def _():
                    issue_gather(obuf, 1, rows_b, sem_b)

        @pl.when(blk + 2 < NB)
        def _():
            fetch_idx(blk + 2, buf, isem)

    @pl.loop(0, NB // 2)
    def _(jj):
        do_block(2 * jj, buf_a, sem_ia, buf_b, sem_ib)
        do_block(2 * jj + 1, buf_b, sem_ib, buf_a, sem_ia)

    plsc.subcore_barrier()

    @pl.loop(0, pl.cdiv(N_ROWCHUNKS, NS))
    def _(i):
        k = i * NS + s

        @pl.when(k < N_ROWCHUNKS)
        def _():
            pltpu.sync_copy(acc_sh.at[pl.ds(k * ROWCHUNK, ROWCHUNK)],
                            out_hbm.at[c, pl.ds(k * ROWCHUNK, ROWCHUNK)])


@jax.jit
def _segment_sums(x2flat, idx, zeros):
    mesh = plsc.VectorSubcoreMesh(core_axis_name="c", subcore_axis_name="s",
                                  num_cores=NC, num_subcores=NS)
    kern = pl.kernel(
        _seg_kernel,
        out_type=jax.ShapeDtypeStruct((NC, ACC_ROWS, HH), jnp.float32),
        mesh=mesh,
        scratch_types=[
            pltpu.VMEM((BPB, 2, CHUNK), jnp.int32),
            pltpu.VMEM((BPB, 2, CHUNK), jnp.int32),
            pltpu.VMEM((CHUNK, HH), jnp.float32),
            pltpu.VMEM((CHUNK, HH), jnp.float32),
            pltpu.VMEM_SHARED((ACC_ROWS, HH), jnp.float32),
            pltpu.SemaphoreType.DMA,
            pltpu.SemaphoreType.DMA,
            pltpu.SemaphoreType.DMA,
            pltpu.SemaphoreType.DMA,
        ],
        compiler_params=pltpu.CompilerParams(use_tc_tiling_on_sc=False),
    )
    return kern(x2flat, idx, zeros)


def _dense_kernel(t_ref, x_ref, acc_ref, rel_ref, W_ref, b_ref,
                  lns_ref, lnb_ref, out_ref):
    j = pl.program_id(0)
    acc = acc_ref[...]
    rel = rel_ref[...]

    agg_lo = (acc[0, 0] * rel[0, :HH] + acc[0, 1] * rel[1, :HH]
              + acc[0, 2] * rel[2, :HH])
    agg_hi = (acc[1, 0] * rel[0, HH:] + acc[1, 1] * rel[1, HH:]
              + acc[1, 2] * rel[2, HH:])

    t = t_ref[0]
    row = t - j * BN
    ids = lax.broadcasted_iota(jnp.int32, (BN, 1), 0)
    bmask = (ids == row).astype(jnp.float32)
    agg_lo = agg_lo + bmask
    agg_hi = agg_hi + bmask

    x_lo = x_ref[0]
    x_hi = x_ref[1]
    W = W_ref[...]

    dot = functools.partial(jax.lax.dot_general,
                            dimension_numbers=(((1,), (0,)), ((), ())),
                            preferred_element_type=jnp.float32)
    h = (dot(x_lo, W[0:HH]) + dot(x_hi, W[HH:H])
         + dot(agg_lo, W[H:H + HH]) + dot(agg_hi, W[H + HH:])
         + b_ref[...])

    mean = jnp.mean(h, axis=-1, keepdims=True)
    var = jnp.mean((h - mean) ** 2, axis=-1, keepdims=True)
    h = (h - mean) * lax.rsqrt(var + 1e-5) * lns_ref[...] + lnb_ref[...]
    h = jnp.maximum(h, 0.0)

    out_ref[0] = h[:, :HH] + x_lo
    out_ref[1] = h[:, HH:] + x_hi


@jax.jit
def _dense_layer(t, x2, acc, rel, W, b, lns, lnb):
    acc4 = acc.reshape(NC, R, STRIDE, HH)
    return pl.pallas_call(
        _dense_kernel,
        grid=(GRID,),
        in_specs=[
            pl.BlockSpec(memory_space=pltpu.SMEM),
            pl.BlockSpec((NC, BN, HH), lambda j: (0, j, 0)),
            pl.BlockSpec((NC, R, BN, HH), lambda j: (0, 0, j, 0)),
            pl.BlockSpec((R, H), lambda j: (0, 0)),
            pl.BlockSpec((2 * H, H), lambda j: (0, 0)),
            pl.BlockSpec((1, H), lambda j: (0, 0)),
            pl.BlockSpec((1, H), lambda j: (0, 0)),
            pl.BlockSpec((1, H), lambda j: (0, 0)),
        ],
        out_specs=pl.BlockSpec((NC, BN, HH), lambda j: (0, j, 0)),
        out_shape=jax.ShapeDtypeStruct((NC, N, HH), jnp.float32),
    )(t, x2, acc4, rel, W, b[None], lns[None], lnb[None])


def kernel(edge_index, edge_type, target_token_ids, rel_emb, W, b,
           ln_scale, ln_bias):
    src = edge_index[0]
    dst = edge_index[1]
    t = target_token_ids[0]

    pad = EP - E
    src_p = jnp.concatenate([src, jnp.full((pad,), 2 * N, jnp.int32)])
    off = jnp.concatenate([jnp.full((E,), N, jnp.int32),
                           jnp.zeros((pad,), jnp.int32)])
    src2 = jnp.stack([src_p, src_p + off]).reshape(2, NS, NB, BPB, 1, CHUNK)
    cidx = jnp.concatenate([
        edge_type * STRIDE + dst,
        jnp.zeros((pad,), jnp.int32),
    ]).reshape(1, NS, NB, BPB, 1, CHUNK)
    idx = jnp.concatenate(
        [src2, jnp.broadcast_to(cidx, src2.shape)], axis=4)
    zeros = jnp.zeros((ROWCHUNK, HH), jnp.float32)

    x2 = jnp.zeros((NC, N, HH), jnp.float32).at[:, t, :].set(1.0)
    tt = t.reshape(1).astype(jnp.int32)

    zrows = jnp.zeros((NXR, HH), jnp.float32)
    for l in range(L):
        x2flat = jnp.concatenate([x2.reshape(NC * N, HH), zrows])
        acc = _segment_sums(x2flat, idx, zeros)
        x2 = _dense_layer(tt, x2, acc, rel_emb[l], W[l], b[l],
                          ln_scale[l], ln_bias[l])

    return jnp.concatenate([x2[0], x2[1]], axis=-1)[None]


# speedup vs baseline: 1.0014x; 1.0014x over previous
import functools

import jax
import jax.numpy as jnp
from jax import lax
from jax.experimental import pallas as pl
from jax.experimental.pallas import tpu as pltpu
from jax.experimental.pallas import tpu_sc as plsc

N = 10000
E = 320000
H = 128
HH = H // 2
L = 4
R = 3

NC = 2
NS = 16

STRIDE = 10000
ACC_ROWS = R * STRIDE
ROWCHUNK = 80
N_ROWCHUNKS = ACC_ROWS // ROWCHUNK
CHUNK = 64
EP_TILE = 20480
EP = NS * EP_TILE
N_CHUNKS = EP_TILE // CHUNK
BPB = 4
NB = N_CHUNKS // BPB
NXR = 8

BN = 80
GRID = N // BN


def _seg_kernel(x_hbm, idx_hbm, zeros_hbm, out_hbm,
                buf_a, buf_b, rows_a, rows_b, acc_sh,
                sem_ia, sem_ib, sem_a, sem_b):
    c = lax.axis_index("c")
    s = lax.axis_index("s")

    def fetch_idx(blk, buf, isem):
        pltpu.async_copy(idx_hbm.at[c, s, blk], buf, isem)

    def wait_idx(blk, buf, isem):
        pltpu.make_async_copy(idx_hbm.at[c, s, blk], buf, isem).wait()

    def issue_gather(buf, u, rows, sem):
        pltpu.async_copy(x_hbm.at[buf.at[u, 0]], rows, sem)

    def wait_gather(buf, u, rows, sem):
        pltpu.make_async_copy(x_hbm.at[buf.at[u, 0]], rows, sem).wait()

    def issue_scat(buf, u, rows, sem):
        pltpu.async_copy(rows, acc_sh.at[buf.at[u, 1]], sem, add=True)

    def wait_scat(buf, u, rows, sem):
        pltpu.make_async_copy(rows, acc_sh.at[buf.at[u, 1]], sem).wait()

    fetch_idx(0, buf_a, sem_ia)
    fetch_idx(1, buf_b, sem_ib)

    @pl.loop(0, pl.cdiv(N_ROWCHUNKS, NS))
    def _(i):
        k = i * NS + s

        @pl.when(k < N_ROWCHUNKS)
        def _():
            pltpu.sync_copy(zeros_hbm, acc_sh.at[pl.ds(k * ROWCHUNK, ROWCHUNK)])

    wait_idx(0, buf_a, sem_ia)
    issue_gather(buf_a, 0, rows_a, sem_a)
    issue_gather(buf_a, 1, rows_b, sem_b)
    plsc.subcore_barrier()

    def do_block(blk, buf, isem, obuf, oisem):
        for p in range(BPB // 2):
            u = 2 * p
            wait_gather(buf, u, rows_a, sem_a)
            issue_scat(buf, u, rows_a, sem_a)
            wait_gather(buf, u + 1, rows_b, sem_b)
            issue_scat(buf, u + 1, rows_b, sem_b)
            wait_scat(buf, u, rows_a, sem_a)
            if u + 2 < BPB:
                issue_gather(buf, u + 2, rows_a, sem_a)
            else:
                @pl.when(blk + 1 < NB)
                def _():
                    wait_idx(blk + 1, obuf, oisem)
                    issue_gather(obuf, 0, rows_a, sem_a)
            wait_scat(buf, u + 1, rows_b, sem_b)
            if u + 3 < BPB:
                issue_gather(buf, u + 3, rows_b, sem_b)
            else:
                @pl.when(blk + 1 < NB)
                def _():
                    issue_gather(obuf, 1, rows_b, sem_b)

        @pl.when(blk + 2 < NB)
        def _():
            fetch_idx(blk + 2, buf, isem)

    @pl.loop(0, NB // 2)
    def _(jj):
        do_block(2 * jj, buf_a, sem_ia, buf_b, sem_ib)
        do_block(2 * jj + 1, buf_b, sem_ib, buf_a, sem_ia)

    plsc.subcore_barrier()

    @pl.loop(0, pl.cdiv(N_ROWCHUNKS, NS))
    def _(i):
        k = i * NS + s

        @pl.when(k < N_ROWCHUNKS)
        def _():
            pltpu.sync_copy(acc_sh.at[pl.ds(k * ROWCHUNK, ROWCHUNK)],
                            out_hbm.at[c, pl.ds(k * ROWCHUNK, ROWCHUNK)])


@jax.jit
def _segment_sums(x2flat, idx, zeros):
    mesh = plsc.VectorSubcoreMesh(core_axis_name="c", subcore_axis_name="s",
                                  num_cores=NC, num_subcores=NS)
    kern = pl.kernel(
        _seg_kernel,
        out_type=jax.ShapeDtypeStruct((NC, ACC_ROWS, HH), jnp.float32),
        mesh=mesh,
        scratch_types=[
            pltpu.VMEM((BPB, 2, CHUNK), jnp.int32),
            pltpu.VMEM((BPB, 2, CHUNK), jnp.int32),
            pltpu.VMEM((CHUNK, HH), jnp.float32),
            pltpu.VMEM((CHUNK, HH), jnp.float32),
            pltpu.VMEM_SHARED((ACC_ROWS, HH), jnp.float32),
            pltpu.SemaphoreType.DMA,
            pltpu.SemaphoreType.DMA,
            pltpu.SemaphoreType.DMA,
            pltpu.SemaphoreType.DMA,
        ],
        compiler_params=pltpu.CompilerParams(use_tc_tiling_on_sc=False),
    )
    return kern(x2flat, idx, zeros)


def _dense_kernel(t_ref, x_ref, acc_ref, rel_ref, W_ref, b_ref,
                  lns_ref, lnb_ref, out_ref):
    j = pl.program_id(0)
    acc = acc_ref[...]
    rel = rel_ref[...]

    agg_lo = (acc[0, 0] * rel[0, :HH] + acc[0, 1] * rel[1, :HH]
              + acc[0, 2] * rel[2, :HH])
    agg_hi = (acc[1, 0] * rel[0, HH:] + acc[1, 1] * rel[1, HH:]
              + acc[1, 2] * rel[2, HH:])

    t = t_ref[0]
    row = t - j * BN
    ids = lax.broadcasted_iota(jnp.int32, (BN, 1), 0)
    bmask = (ids == row).astype(jnp.float32)
    agg_lo = agg_lo + bmask
    agg_hi = agg_hi + bmask

    x_lo = x_ref[0]
    x_hi = x_ref[1]
    W = W_ref[...]

    dot = functools.partial(jax.lax.dot_general,
                            dimension_numbers=(((1,), (0,)), ((), ())),
                            preferred_element_type=jnp.float32)
    h = (dot(x_lo, W[0:HH]) + dot(x_hi, W[HH:H])
         + dot(agg_lo, W[H:H + HH]) + dot(agg_hi, W[H + HH:])
         + b_ref[...])

    mean = jnp.mean(h, axis=-1, keepdims=True)
    var = jnp.mean((h - mean) ** 2, axis=-1, keepdims=True)
    h = (h - mean) * lax.rsqrt(var + 1e-5) * lns_ref[...] + lnb_ref[...]
    h = jnp.maximum(h, 0.0)

    out_ref[0] = h[:, :HH] + x_lo
    out_ref[1] = h[:, HH:] + x_hi


@jax.jit
def _dense_layer(t, x2, acc, rel, W, b, lns, lnb):
    acc4 = acc.reshape(NC, R, STRIDE, HH)
    return pl.pallas_call(
        _dense_kernel,
        grid=(GRID,),
        in_specs=[
            pl.BlockSpec(memory_space=pltpu.SMEM),
            pl.BlockSpec((NC, BN, HH), lambda j: (0, j, 0)),
            pl.BlockSpec((NC, R, BN, HH), lambda j: (0, 0, j, 0)),
            pl.BlockSpec((R, H), lambda j: (0, 0)),
            pl.BlockSpec((2 * H, H), lambda j: (0, 0)),
            pl.BlockSpec((1, H), lambda j: (0, 0)),
            pl.BlockSpec((1, H), lambda j: (0, 0)),
            pl.BlockSpec((1, H), lambda j: (0, 0)),
        ],
        out_specs=pl.BlockSpec((NC, BN, HH), lambda j: (0, j, 0)),
        out_shape=jax.ShapeDtypeStruct((NC, N, HH), jnp.float32),
    )(t, x2, acc4, rel, W, b[None], lns[None], lnb[None])


def kernel(edge_index, edge_type, target_token_ids, rel_emb, W, b,
           ln_scale, ln_bias):
    src = edge_index[0]
    dst = edge_index[1]
    t = target_token_ids[0]

    pad = EP - E
    src_p = jnp.concatenate([src, jnp.full((pad,), 2 * N, jnp.int32)])
    off = jnp.concatenate([jnp.full((E,), N, jnp.int32),
                           jnp.zeros((pad,), jnp.int32)])
    src2 = jnp.stack([src_p, src_p + off]).reshape(2, NS, NB, BPB, 1, CHUNK)
    cidx = jnp.concatenate([
        edge_type * STRIDE + dst,
        jnp.zeros((pad,), jnp.int32),
    ]).reshape(1, NS, NB, BPB, 1, CHUNK)
    idx = jnp.concatenate(
        [src2, jnp.broadcast_to(cidx, src2.shape)], axis=4)
    zeros = jnp.zeros((ROWCHUNK, HH), jnp.float32)

    x2 = jnp.zeros((NC, N, HH), jnp.float32).at[:, t, :].set(1.0)
    tt = t.reshape(1).astype(jnp.int32)

    zrows = jnp.zeros((NXR, HH), jnp.float32)
    for l in range(L):
        x2flat = jnp.concatenate([x2.reshape(NC * N, HH), zrows])
        acc = _segment_sums(x2flat, idx, zeros)
        x2 = _dense_layer(tt, x2, acc, rel_emb[l], W[l], b[l],
                          ln_scale[l], ln_bias[l])

    return jnp.concatenate([x2[0], x2[1]], axis=-1)[None]


# speedup vs baseline: 1.1418x; 1.1402x over previous
import functools

import jax
import jax.numpy as jnp
from jax import lax
from jax.experimental import pallas as pl
from jax.experimental.pallas import tpu as pltpu
from jax.experimental.pallas import tpu_sc as plsc

N = 10000
E = 320000
H = 128
HH = H // 2
L = 4
R = 3

NC = 2
NS = 16

STRIDE = 10080
ACC_ROWS = R * STRIDE
ROWCHUNK = 80
N_ROWCHUNKS = ACC_ROWS // ROWCHUNK
CHUNK = 64
EP_TILE = 20480
EP = NS * EP_TILE
N_CHUNKS = EP_TILE // CHUNK
BPB = 4
NB = N_CHUNKS // BPB

BN = 80
GRID = N // BN


def _seg_kernel(x_hbm, idx_hbm, zeros_hbm, out_hbm,
                buf_a, buf_b, rows_a, rows_b, acc_sh,
                sem_ia, sem_ib, sem_a, sem_b):
    c = lax.axis_index("c")
    s = lax.axis_index("s")

    def fetch_idx(blk, buf, isem):
        pltpu.async_copy(idx_hbm.at[c, s, blk], buf, isem)

    def wait_idx(blk, buf, isem):
        pltpu.make_async_copy(idx_hbm.at[c, s, blk], buf, isem).wait()

    def issue_gather(buf, u, rows, sem):
        pltpu.async_copy(x_hbm.at[buf.at[u, 0]], rows, sem)

    def wait_gather(buf, u, rows, sem):
        pltpu.make_async_copy(x_hbm.at[buf.at[u, 0]], rows, sem).wait()

    def issue_scat(buf, u, rows, sem):
        pltpu.async_copy(rows, acc_sh.at[buf.at[u, 1]], sem, add=True)

    def wait_scat(buf, u, rows, sem):
        pltpu.make_async_copy(rows, acc_sh.at[buf.at[u, 1]], sem).wait()

    fetch_idx(0, buf_a, sem_ia)
    fetch_idx(1, buf_b, sem_ib)

    @pl.loop(0, pl.cdiv(N_ROWCHUNKS, NS))
    def _(i):
        k = i * NS + s

        @pl.when(k < N_ROWCHUNKS)
        def _():
            pltpu.sync_copy(zeros_hbm, acc_sh.at[pl.ds(k * ROWCHUNK, ROWCHUNK)])

    wait_idx(0, buf_a, sem_ia)
    issue_gather(buf_a, 0, rows_a, sem_a)
    issue_gather(buf_a, 1, rows_b, sem_b)
    plsc.subcore_barrier()

    def do_block(blk, buf, isem, obuf, oisem):
        for p in range(BPB // 2):
            u = 2 * p
            wait_gather(buf, u, rows_a, sem_a)
            issue_scat(buf, u, rows_a, sem_a)
            wait_gather(buf, u + 1, rows_b, sem_b)
            issue_scat(buf, u + 1, rows_b, sem_b)
            wait_scat(buf, u, rows_a, sem_a)
            if u + 2 < BPB:
                issue_gather(buf, u + 2, rows_a, sem_a)
            else:
                @pl.when(blk + 1 < NB)
                def _():
                    wait_idx(blk + 1, obuf, oisem)
                    issue_gather(obuf, 0, rows_a, sem_a)
            wait_scat(buf, u + 1, rows_b, sem_b)
            if u + 3 < BPB:
                issue_gather(buf, u + 3, rows_b, sem_b)
            else:
                @pl.when(blk + 1 < NB)
                def _():
                    issue_gather(obuf, 1, rows_b, sem_b)

        @pl.when(blk + 2 < NB)
        def _():
            fetch_idx(blk + 2, buf, isem)

    @pl.loop(0, NB // 2)
    def _(jj):
        do_block(2 * jj, buf_a, sem_ia, buf_b, sem_ib)
        do_block(2 * jj + 1, buf_b, sem_ib, buf_a, sem_ia)

    plsc.subcore_barrier()

    @pl.loop(0, pl.cdiv(N_ROWCHUNKS, NS))
    def _(i):
        k = i * NS + s

        @pl.when(k < N_ROWCHUNKS)
        def _():
            pltpu.sync_copy(acc_sh.at[pl.ds(k * ROWCHUNK, ROWCHUNK)],
                            out_hbm.at[c, pl.ds(k * ROWCHUNK, ROWCHUNK)])


@jax.jit
def _segment_sums(x2flat, idx, zeros):
    mesh = plsc.VectorSubcoreMesh(core_axis_name="c", subcore_axis_name="s",
                                  num_cores=NC, num_subcores=NS)
    kern = pl.kernel(
        _seg_kernel,
        out_type=jax.ShapeDtypeStruct((NC, ACC_ROWS, HH), jnp.float32),
        mesh=mesh,
        scratch_types=[
            pltpu.VMEM((BPB, 2, CHUNK), jnp.int32),
            pltpu.VMEM((BPB, 2, CHUNK), jnp.int32),
            pltpu.VMEM((CHUNK, HH), jnp.float32),
            pltpu.VMEM((CHUNK, HH), jnp.float32),
            pltpu.VMEM_SHARED((ACC_ROWS, HH), jnp.float32),
            pltpu.SemaphoreType.DMA,
            pltpu.SemaphoreType.DMA,
            pltpu.SemaphoreType.DMA,
            pltpu.SemaphoreType.DMA,
        ],
        compiler_params=pltpu.CompilerParams(use_tc_tiling_on_sc=False),
    )
    return kern(x2flat, idx, zeros)


def _dense_kernel(t_ref, x_ref, acc_ref, rel_ref, W_ref, b_ref,
                  lns_ref, lnb_ref, out_ref):
    j = pl.program_id(0)
    acc = acc_ref[...]
    rel = rel_ref[...]

    agg_lo = (acc[0, 0] * rel[0, :HH] + acc[0, 1] * rel[1, :HH]
              + acc[0, 2] * rel[2, :HH])
    agg_hi = (acc[1, 0] * rel[0, HH:] + acc[1, 1] * rel[1, HH:]
              + acc[1, 2] * rel[2, HH:])

    t = t_ref[0]
    row = t - j * BN
    ids = lax.broadcasted_iota(jnp.int32, (BN, 1), 0)
    bmask = (ids == row).astype(jnp.float32)
    agg_lo = agg_lo + bmask
    agg_hi = agg_hi + bmask

    x_lo = x_ref[0]
    x_hi = x_ref[1]
    W = W_ref[...]

    dot = functools.partial(jax.lax.dot_general,
                            dimension_numbers=(((1,), (0,)), ((), ())),
                            preferred_element_type=jnp.float32)
    h = (dot(x_lo, W[0:HH]) + dot(x_hi, W[HH:H])
         + dot(agg_lo, W[H:H + HH]) + dot(agg_hi, W[H + HH:])
         + b_ref[...])

    mean = jnp.mean(h, axis=-1, keepdims=True)
    var = jnp.mean((h - mean) ** 2, axis=-1, keepdims=True)
    h = (h - mean) * lax.rsqrt(var + 1e-5) * lns_ref[...] + lnb_ref[...]
    h = jnp.maximum(h, 0.0)

    out_ref[0] = h[:, :HH] + x_lo
    out_ref[1] = h[:, HH:] + x_hi


@jax.jit
def _dense_layer(t, x2, acc, rel, W, b, lns, lnb):
    acc4 = acc.reshape(NC, R, STRIDE, HH)
    return pl.pallas_call(
        _dense_kernel,
        grid=(GRID,),
        in_specs=[
            pl.BlockSpec(memory_space=pltpu.SMEM),
            pl.BlockSpec((NC, BN, HH), lambda j: (0, j, 0)),
            pl.BlockSpec((NC, R, BN, HH), lambda j: (0, 0, j, 0)),
            pl.BlockSpec((R, H), lambda j: (0, 0)),
            pl.BlockSpec((2 * H, H), lambda j: (0, 0)),
            pl.BlockSpec((1, H), lambda j: (0, 0)),
            pl.BlockSpec((1, H), lambda j: (0, 0)),
            pl.BlockSpec((1, H), lambda j: (0, 0)),
        ],
        out_specs=pl.BlockSpec((NC, BN, HH), lambda j: (0, j, 0)),
        out_shape=jax.ShapeDtypeStruct((NC, N, HH), jnp.float32),
    )(t, x2, acc4, rel, W, b[None], lns[None], lnb[None])


def kernel(edge_index, edge_type, target_token_ids, rel_emb, W, b,
           ln_scale, ln_bias):
    src = edge_index[0]
    dst = edge_index[1]
    t = target_token_ids[0]

    pad = EP - E
    src_p = jnp.concatenate([src, jnp.zeros((pad,), jnp.int32)])
    src2 = jnp.stack([src_p, src_p + N]).reshape(2, NS, NB, BPB, 1, CHUNK)
    cidx = jnp.concatenate([
        edge_type * STRIDE + dst,
        jnp.full((pad,), N, jnp.int32),
    ]).reshape(1, NS, NB, BPB, 1, CHUNK)
    idx = jnp.concatenate(
        [src2, jnp.broadcast_to(cidx, src2.shape)], axis=4)
    zeros = jnp.zeros((ROWCHUNK, HH), jnp.float32)

    x2 = jnp.zeros((NC, N, HH), jnp.float32).at[:, t, :].set(1.0)
    tt = t.reshape(1).astype(jnp.int32)

    for l in range(L):
        acc = _segment_sums(x2.reshape(NC * N, HH), idx, zeros)
        x2 = _dense_layer(tt, x2, acc, rel_emb[l], W[l], b[l],
                          ln_scale[l], ln_bias[l])

    return jnp.concatenate([x2[0], x2[1]], axis=-1)[None]


# speedup vs baseline: 1.3759x; 1.2050x over previous
import functools

import jax
import jax.numpy as jnp
from jax import lax
from jax.experimental import pallas as pl
from jax.experimental.pallas import tpu as pltpu
from jax.experimental.pallas import tpu_sc as plsc

N = 10000
E = 320000
H = 128
HH = H // 2
L = 4
R = 3

NC = 2
NS = 16

STRIDE = 10080
ACC_ROWS = R * STRIDE
ROWCHUNK = 80
N_ROWCHUNKS = ACC_ROWS // ROWCHUNK
CHUNK = 64
EP_TILE = 20480
EP = NS * EP_TILE
N_CHUNKS = EP_TILE // CHUNK
BPB = 4
NB = N_CHUNKS // BPB

BN = 80
GRID = N // BN


def _seg_kernel(x_hbm, idx_hbm, zeros_hbm, out_hbm,
                buf_a, buf_b, rows_a, rows_b, acc_sh,
                sem_ia, sem_ib, sem_a, sem_b):
    c = lax.axis_index("c")
    s = lax.axis_index("s")

    def fetch_idx(blk, buf, isem):
        pltpu.async_copy(idx_hbm.at[c, s, blk], buf, isem)

    def wait_idx(blk, buf, isem):
        pltpu.make_async_copy(idx_hbm.at[c, s, blk], buf, isem).wait()

    def issue_gather(buf, u, rows, sem):
        pltpu.async_copy(x_hbm.at[buf.at[u, 0]], rows, sem)

    def wait_gather(buf, u, rows, sem):
        pltpu.make_async_copy(x_hbm.at[buf.at[u, 0]], rows, sem).wait()

    def issue_scat(buf, u, rows, sem):
        pltpu.async_copy(rows, acc_sh.at[buf.at[u, 1]], sem, add=True)

    def wait_scat(buf, u, rows, sem):
        pltpu.make_async_copy(rows, acc_sh.at[buf.at[u, 1]], sem).wait()

    fetch_idx(0, buf_a, sem_ia)
    fetch_idx(1, buf_b, sem_ib)

    @pl.loop(0, pl.cdiv(N_ROWCHUNKS, NS))
    def _(i):
        k = i * NS + s

        @pl.when(k < N_ROWCHUNKS)
        def _():
            pltpu.sync_copy(zeros_hbm, acc_sh.at[pl.ds(k * ROWCHUNK, ROWCHUNK)])

    wait_idx(0, buf_a, sem_ia)
    issue_gather(buf_a, 0, rows_a, sem_a)
    issue_gather(buf_a, 1, rows_b, sem_b)
    plsc.subcore_barrier()

    def do_block(blk, buf, isem, obuf, oisem):
        for p in range(BPB // 2):
            u = 2 * p
            wait_gather(buf, u, rows_a, sem_a)
            issue_scat(buf, u, rows_a, sem_a)
            wait_gather(buf, u + 1, rows_b, sem_b)
            issue_scat(buf, u + 1, rows_b, sem_b)
            wait_scat(buf, u, rows_a, sem_a)
            if u + 2 < BPB:
                issue_gather(buf, u + 2, rows_a, sem_a)
            else:
                @pl.when(blk + 1 < NB)
                def _():
                    wait_idx(blk + 1, obuf, oisem)
                    issue_gather(obuf, 0, rows_a, sem_a)
            wait_scat(buf, u + 1, rows_b, sem_b)
            if u + 3 < BPB:
                issue_gather(buf, u + 3, rows_b, sem_b)
            else:
                @pl.when(blk + 1 < NB)
                def _():
                    issue_gather(obuf, 1, rows_b, sem_b)

        @pl.when(blk + 2 < NB)
        def _():
            fetch_idx(blk + 2, buf, isem)

    @pl.loop(0, NB // 2)
    def _(jj):
        do_block(2 * jj, buf_a, sem_ia, buf_b, sem_ib)
        do_block(2 * jj + 1, buf_b, sem_ib, buf_a, sem_ia)

    plsc.subcore_barrier()

    @pl.loop(0, pl.cdiv(N_ROWCHUNKS, NS))
    def _(i):
        k = i * NS + s

        @pl.when(k < N_ROWCHUNKS)
        def _():
            pltpu.sync_copy(acc_sh.at[pl.ds(k * ROWCHUNK, ROWCHUNK)],
                            out_hbm.at[c, pl.ds(k * ROWCHUNK, ROWCHUNK)])


@jax.jit
def _segment_sums(x2flat, idx, zeros):
    mesh = plsc.VectorSubcoreMesh(core_axis_name="c", subcore_axis_name="s",
                                  num_cores=NC, num_subcores=NS)
    kern = pl.kernel(
        _seg_kernel,
        out_type=jax.ShapeDtypeStruct((NC, ACC_ROWS, HH), jnp.float32),
        mesh=mesh,
        scratch_types=[
            pltpu.VMEM((BPB, 2, CHUNK), jnp.int32),
            pltpu.VMEM((BPB, 2, CHUNK), jnp.int32),
            pltpu.VMEM((CHUNK, HH), jnp.float32),
            pltpu.VMEM((CHUNK, HH), jnp.float32),
            pltpu.VMEM_SHARED((ACC_ROWS, HH), jnp.float32),
            pltpu.SemaphoreType.DMA,
            pltpu.SemaphoreType.DMA,
            pltpu.SemaphoreType.DMA,
            pltpu.SemaphoreType.DMA,
        ],
        compiler_params=pltpu.CompilerParams(use_tc_tiling_on_sc=False),
    )
    return kern(x2flat, idx, zeros)


NBH = NB // 2


def _count_kernel(idx_hbm, t_hbm, out_hbm, ibuf, cnt, t_vmem, sem):
    c = lax.axis_index("c")
    s = lax.axis_index("s")
    pltpu.async_copy(t_hbm, t_vmem, sem).wait()
    tcmp = t_vmem[...] + c * N

    @pl.loop(0, ACC_ROWS // 16)
    def _(i):
        cnt[pl.ds(i * 16, 16)] = jnp.zeros((16,), jnp.int32)

    ones = jnp.ones((16,), jnp.int32)

    @pl.loop(0, NBH)
    def _(j):
        pltpu.sync_copy(idx_hbm.at[c, s, c * NBH + j], ibuf)
        for u in range(BPB):
            @pl.loop(0, CHUNK // 16)
            def _(g):
                sv = ibuf[u, 0, pl.ds(g * 16, 16)]
                cv = ibuf[u, 1, pl.ds(g * 16, 16)]
                plsc.addupdate_scatter(cnt, [cv], ones, mask=sv == tcmp)

    pltpu.sync_copy(cnt, out_hbm.at[c, s])


@jax.jit
def _edge_counts(idx, tt):
    mesh = plsc.VectorSubcoreMesh(core_axis_name="c", subcore_axis_name="s",
                                  num_cores=NC, num_subcores=NS)
    kern = pl.kernel(
        _count_kernel,
        out_type=jax.ShapeDtypeStruct((NC, NS, ACC_ROWS), jnp.int32),
        mesh=mesh,
        scratch_types=[
            pltpu.VMEM((BPB, 2, CHUNK), jnp.int32),
            pltpu.VMEM((ACC_ROWS,), jnp.int32),
            pltpu.VMEM((16,), jnp.int32),
            pltpu.SemaphoreType.DMA,
        ],
        compiler_params=pltpu.CompilerParams(use_tc_tiling_on_sc=False,
                                             needs_layout_passes=False),
    )
    return kern(idx, tt)


def _dense_kernel(t_ref, x_ref, acc_ref, rel_ref, W_ref, b_ref,
                  lns_ref, lnb_ref, out_ref):
    j = pl.program_id(0)
    acc = acc_ref[...]
    rel = rel_ref[...]

    agg_lo = (acc[0, 0] * rel[0, :HH] + acc[0, 1] * rel[1, :HH]
              + acc[0, 2] * rel[2, :HH])
    agg_hi = (acc[1, 0] * rel[0, HH:] + acc[1, 1] * rel[1, HH:]
              + acc[1, 2] * rel[2, HH:])

    t = t_ref[0]
    row = t - j * BN
    ids = lax.broadcasted_iota(jnp.int32, (BN, 1), 0)
    bmask = (ids == row).astype(jnp.float32)
    agg_lo = agg_lo + bmask
    agg_hi = agg_hi + bmask

    x_lo = x_ref[0]
    x_hi = x_ref[1]
    W = W_ref[...]

    dot = functools.partial(jax.lax.dot_general,
                            dimension_numbers=(((1,), (0,)), ((), ())),
                            preferred_element_type=jnp.float32)
    h = (dot(x_lo, W[0:HH]) + dot(x_hi, W[HH:H])
         + dot(agg_lo, W[H:H + HH]) + dot(agg_hi, W[H + HH:])
         + b_ref[...])

    mean = jnp.mean(h, axis=-1, keepdims=True)
    var = jnp.mean((h - mean) ** 2, axis=-1, keepdims=True)
    h = (h - mean) * lax.rsqrt(var + 1e-5) * lns_ref[...] + lnb_ref[...]
    h = jnp.maximum(h, 0.0)

    out_ref[0] = h[:, :HH] + x_lo
    out_ref[1] = h[:, HH:] + x_hi


@jax.jit
def _dense_layer(t, x2, acc, rel, W, b, lns, lnb):
    acc4 = acc.reshape(NC, R, STRIDE, HH)
    return pl.pallas_call(
        _dense_kernel,
        grid=(GRID,),
        in_specs=[
            pl.BlockSpec(memory_space=pltpu.SMEM),
            pl.BlockSpec((NC, BN, HH), lambda j: (0, j, 0)),
            pl.BlockSpec((NC, R, BN, HH), lambda j: (0, 0, j, 0)),
            pl.BlockSpec((R, H), lambda j: (0, 0)),
            pl.BlockSpec((2 * H, H), lambda j: (0, 0)),
            pl.BlockSpec((1, H), lambda j: (0, 0)),
            pl.BlockSpec((1, H), lambda j: (0, 0)),
            pl.BlockSpec((1, H), lambda j: (0, 0)),
        ],
        out_specs=pl.BlockSpec((NC, BN, HH), lambda j: (0, j, 0)),
        out_shape=jax.ShapeDtypeStruct((NC, N, HH), jnp.float32),
    )(t, x2, acc4, rel, W, b[None], lns[None], lnb[None])


def kernel(edge_index, edge_type, target_token_ids, rel_emb, W, b,
           ln_scale, ln_bias):
    src = edge_index[0]
    dst = edge_index[1]
    t = target_token_ids[0]

    pad = EP - E
    src_p = jnp.concatenate([src, jnp.zeros((pad,), jnp.int32)])
    src2 = jnp.stack([src_p, src_p + N]).reshape(2, NS, NB, BPB, 1, CHUNK)
    cidx = jnp.concatenate([
        edge_type * STRIDE + dst,
        jnp.full((pad,), N, jnp.int32),
    ]).reshape(1, NS, NB, BPB, 1, CHUNK)
    idx = jnp.concatenate(
        [src2, jnp.broadcast_to(cidx, src2.shape)], axis=4)
    zeros = jnp.zeros((ROWCHUNK, HH), jnp.float32)

    x2 = jnp.zeros((NC, N, HH), jnp.float32).at[:, t, :].set(1.0)
    tt = t.reshape(1).astype(jnp.int32)

    for l in range(L):
        if l == 0:
            tt16 = jnp.full((16,), t, jnp.int32)
            cnts = _edge_counts(idx, tt16).sum(axis=(0, 1)).astype(jnp.float32)
            acc = jnp.broadcast_to(cnts[None, :, None], (NC, ACC_ROWS, HH))
        else:
            acc = _segment_sums(x2.reshape(NC * N, HH), idx, zeros)
        x2 = _dense_layer(tt, x2, acc, rel_emb[l], W[l], b[l],
                          ln_scale[l], ln_bias[l])

    return jnp.concatenate([x2[0], x2[1]], axis=-1)[None]


# speedup vs baseline: 1.5581x; 1.1324x over previous
import functools

import jax
import jax.numpy as jnp
from jax import lax
from jax.experimental import pallas as pl
from jax.experimental.pallas import tpu as pltpu
from jax.experimental.pallas import tpu_sc as plsc

N = 10000
E = 320000
H = 128
HH = H // 2
L = 4
R = 3

NC = 2
NS = 16

STRIDE = 10080
ACC_ROWS = R * STRIDE
ROWCHUNK = 80
N_ROWCHUNKS = ACC_ROWS // ROWCHUNK
CHUNK = 64
EP_TILE = 20480
EP = NS * EP_TILE
N_CHUNKS = EP_TILE // CHUNK
BPB = 4
NB = N_CHUNKS // BPB

BN = 1000
GRID = N // BN


def _seg_kernel(x_hbm, idx_hbm, zeros_hbm, out_hbm,
                buf_a, buf_b, rows_a, rows_b, acc_sh,
                sem_ia, sem_ib, sem_a, sem_b):
    c = lax.axis_index("c")
    s = lax.axis_index("s")

    def fetch_idx(blk, buf, isem):
        pltpu.async_copy(idx_hbm.at[c, s, blk], buf, isem)

    def wait_idx(blk, buf, isem):
        pltpu.make_async_copy(idx_hbm.at[c, s, blk], buf, isem).wait()

    def issue_gather(buf, u, rows, sem):
        pltpu.async_copy(x_hbm.at[buf.at[u, 0]], rows, sem)

    def wait_gather(buf, u, rows, sem):
        pltpu.make_async_copy(x_hbm.at[buf.at[u, 0]], rows, sem).wait()

    def issue_scat(buf, u, rows, sem):
        pltpu.async_copy(rows, acc_sh.at[buf.at[u, 1]], sem, add=True)

    def wait_scat(buf, u, rows, sem):
        pltpu.make_async_copy(rows, acc_sh.at[buf.at[u, 1]], sem).wait()

    fetch_idx(0, buf_a, sem_ia)
    fetch_idx(1, buf_b, sem_ib)

    @pl.loop(0, pl.cdiv(N_ROWCHUNKS, NS))
    def _(i):
        k = i * NS + s

        @pl.when(k < N_ROWCHUNKS)
        def _():
            pltpu.sync_copy(zeros_hbm, acc_sh.at[pl.ds(k * ROWCHUNK, ROWCHUNK)])

    wait_idx(0, buf_a, sem_ia)
    issue_gather(buf_a, 0, rows_a, sem_a)
    issue_gather(buf_a, 1, rows_b, sem_b)
    plsc.subcore_barrier()

    def do_block(blk, buf, isem, obuf, oisem):
        for p in range(BPB // 2):
            u = 2 * p
            wait_gather(buf, u, rows_a, sem_a)
            issue_scat(buf, u, rows_a, sem_a)
            wait_gather(buf, u + 1, rows_b, sem_b)
            issue_scat(buf, u + 1, rows_b, sem_b)
            wait_scat(buf, u, rows_a, sem_a)
            if u + 2 < BPB:
                issue_gather(buf, u + 2, rows_a, sem_a)
            else:
                @pl.when(blk + 1 < NB)
                def _():
                    wait_idx(blk + 1, obuf, oisem)
                    issue_gather(obuf, 0, rows_a, sem_a)
            wait_scat(buf, u + 1, rows_b, sem_b)
            if u + 3 < BPB:
                issue_gather(buf, u + 3, rows_b, sem_b)
            else:
                @pl.when(blk + 1 < NB)
                def _():
                    issue_gather(obuf, 1, rows_b, sem_b)

        @pl.when(blk + 2 < NB)
        def _():
            fetch_idx(blk + 2, buf, isem)

    @pl.loop(0, NB // 2)
    def _(jj):
        do_block(2 * jj, buf_a, sem_ia, buf_b, sem_ib)
        do_block(2 * jj + 1, buf_b, sem_ib, buf_a, sem_ia)

    plsc.subcore_barrier()

    @pl.loop(0, pl.cdiv(N_ROWCHUNKS, NS))
    def _(i):
        k = i * NS + s

        @pl.when(k < N_ROWCHUNKS)
        def _():
            pltpu.sync_copy(acc_sh.at[pl.ds(k * ROWCHUNK, ROWCHUNK)],
                            out_hbm.at[c, pl.ds(k * ROWCHUNK, ROWCHUNK)])


@jax.jit
def _segment_sums(x2flat, idx, zeros):
    mesh = plsc.VectorSubcoreMesh(core_axis_name="c", subcore_axis_name="s",
                                  num_cores=NC, num_subcores=NS)
    kern = pl.kernel(
        _seg_kernel,
        out_type=jax.ShapeDtypeStruct((NC, ACC_ROWS, HH), jnp.float32),
        mesh=mesh,
        scratch_types=[
            pltpu.VMEM((BPB, 2, CHUNK), jnp.int32),
            pltpu.VMEM((BPB, 2, CHUNK), jnp.int32),
            pltpu.VMEM((CHUNK, HH), jnp.float32),
            pltpu.VMEM((CHUNK, HH), jnp.float32),
            pltpu.VMEM_SHARED((ACC_ROWS, HH), jnp.float32),
            pltpu.SemaphoreType.DMA,
            pltpu.SemaphoreType.DMA,
            pltpu.SemaphoreType.DMA,
            pltpu.SemaphoreType.DMA,
        ],
        compiler_params=pltpu.CompilerParams(use_tc_tiling_on_sc=False),
    )
    return kern(x2flat, idx, zeros)


NBH = NB // 2


def _count_kernel(idx_hbm, t_hbm, out_hbm, ibuf, cnt, t_vmem, sem):
    c = lax.axis_index("c")
    s = lax.axis_index("s")
    pltpu.async_copy(t_hbm, t_vmem, sem).wait()
    tcmp = t_vmem[...] + c * N

    @pl.loop(0, ACC_ROWS // 16)
    def _(i):
        cnt[pl.ds(i * 16, 16)] = jnp.zeros((16,), jnp.int32)

    ones = jnp.ones((16,), jnp.int32)

    @pl.loop(0, NBH)
    def _(j):
        pltpu.sync_copy(idx_hbm.at[c, s, c * NBH + j], ibuf)
        for u in range(BPB):
            @pl.loop(0, CHUNK // 16)
            def _(g):
                sv = ibuf[u, 0, pl.ds(g * 16, 16)]
                cv = ibuf[u, 1, pl.ds(g * 16, 16)]
                plsc.addupdate_scatter(cnt, [cv], ones, mask=sv == tcmp)

    pltpu.sync_copy(cnt, out_hbm.at[c, s])


@jax.jit
def _edge_counts(idx, tt):
    mesh = plsc.VectorSubcoreMesh(core_axis_name="c", subcore_axis_name="s",
                                  num_cores=NC, num_subcores=NS)
    kern = pl.kernel(
        _count_kernel,
        out_type=jax.ShapeDtypeStruct((NC, NS, ACC_ROWS), jnp.int32),
        mesh=mesh,
        scratch_types=[
            pltpu.VMEM((BPB, 2, CHUNK), jnp.int32),
            pltpu.VMEM((ACC_ROWS,), jnp.int32),
            pltpu.VMEM((16,), jnp.int32),
            pltpu.SemaphoreType.DMA,
        ],
        compiler_params=pltpu.CompilerParams(use_tc_tiling_on_sc=False,
                                             needs_layout_passes=False),
    )
    return kern(idx, tt)


def _dense_kernel(t_ref, x_ref, a0_ref, a1_ref, a2_ref, rel_ref, W_ref,
                  b_ref, lns_ref, lnb_ref, out_ref):
    j = pl.program_id(0)
    rel = rel_ref[...]

    agg_lo = (a0_ref[0] * rel[0, :HH] + a1_ref[0] * rel[1, :HH]
              + a2_ref[0] * rel[2, :HH])
    agg_hi = (a0_ref[1] * rel[0, HH:] + a1_ref[1] * rel[1, HH:]
              + a2_ref[1] * rel[2, HH:])

    t = t_ref[0]
    row = t - j * BN
    ids = lax.broadcasted_iota(jnp.int32, (BN, 1), 0)
    bmask = (ids == row).astype(jnp.float32)
    agg_lo = agg_lo + bmask
    agg_hi = agg_hi + bmask

    x_lo = x_ref[0]
    x_hi = x_ref[1]
    W = W_ref[...]

    dot = functools.partial(jax.lax.dot_general,
                            dimension_numbers=(((1,), (0,)), ((), ())),
                            preferred_element_type=jnp.float32)
    h = (dot(x_lo, W[0:HH]) + dot(x_hi, W[HH:H])
         + dot(agg_lo, W[H:H + HH]) + dot(agg_hi, W[H + HH:])
         + b_ref[...])

    mean = jnp.mean(h, axis=-1, keepdims=True)
    var = jnp.mean((h - mean) ** 2, axis=-1, keepdims=True)
    h = (h - mean) * lax.rsqrt(var + 1e-5) * lns_ref[...] + lnb_ref[...]
    h = jnp.maximum(h, 0.0)

    out_ref[0] = h[:, :HH] + x_lo
    out_ref[1] = h[:, HH:] + x_hi


@jax.jit
def _dense_layer(t, x2, acc, rel, W, b, lns, lnb):
    a0 = acc[:, 0:N]
    a1 = acc[:, STRIDE:STRIDE + N]
    a2 = acc[:, 2 * STRIDE:2 * STRIDE + N]
    rspec = pl.BlockSpec((NC, BN, HH), lambda j: (0, j, 0))
    return pl.pallas_call(
        _dense_kernel,
        grid=(GRID,),
        in_specs=[
            pl.BlockSpec(memory_space=pltpu.SMEM),
            rspec, rspec, rspec, rspec,
            pl.BlockSpec((R, H), lambda j: (0, 0)),
            pl.BlockSpec((2 * H, H), lambda j: (0, 0)),
            pl.BlockSpec((1, H), lambda j: (0, 0)),
            pl.BlockSpec((1, H), lambda j: (0, 0)),
            pl.BlockSpec((1, H), lambda j: (0, 0)),
        ],
        out_specs=pl.BlockSpec((NC, BN, HH), lambda j: (0, j, 0)),
        out_shape=jax.ShapeDtypeStruct((NC, N, HH), jnp.float32),
    )(t, x2, a0, a1, a2, rel, W, b[None], lns[None], lnb[None])


def kernel(edge_index, edge_type, target_token_ids, rel_emb, W, b,
           ln_scale, ln_bias):
    src = edge_index[0]
    dst = edge_index[1]
    t = target_token_ids[0]

    pad = EP - E
    src_p = jnp.concatenate([src, jnp.zeros((pad,), jnp.int32)])
    src2 = jnp.stack([src_p, src_p + N]).reshape(2, NS, NB, BPB, 1, CHUNK)
    cidx = jnp.concatenate([
        edge_type * STRIDE + dst,
        jnp.full((pad,), N, jnp.int32),
    ]).reshape(1, NS, NB, BPB, 1, CHUNK)
    idx = jnp.concatenate(
        [src2, jnp.broadcast_to(cidx, src2.shape)], axis=4)
    zeros = jnp.zeros((ROWCHUNK, HH), jnp.float32)

    x2 = jnp.zeros((NC, N, HH), jnp.float32).at[:, t, :].set(1.0)
    tt = t.reshape(1).astype(jnp.int32)

    for l in range(L):
        if l == 0:
            tt16 = jnp.full((16,), t, jnp.int32)
            cnts = _edge_counts(idx, tt16).sum(axis=(0, 1)).astype(jnp.float32)
            acc = jnp.broadcast_to(cnts[None, :, None], (NC, ACC_ROWS, HH))
        else:
            acc = _segment_sums(x2.reshape(NC * N, HH), idx, zeros)
        x2 = _dense_layer(tt, x2, acc, rel_emb[l], W[l], b[l],
                          ln_scale[l], ln_bias[l])

    return jnp.concatenate([x2[0], x2[1]], axis=-1)[None]


# speedup vs baseline: 1.6875x; 1.0831x over previous
import functools

import jax
import jax.numpy as jnp
from jax import lax
from jax.experimental import pallas as pl
from jax.experimental.pallas import tpu as pltpu
from jax.experimental.pallas import tpu_sc as plsc

N = 10000
E = 320000
H = 128
HH = H // 2
L = 4
R = 3

NC = 2
NS = 16

STRIDE = 10080
ACC_ROWS = R * STRIDE
ROWCHUNK = 80
N_ROWCHUNKS = ACC_ROWS // ROWCHUNK
CHUNK = 64
EP_TILE = 20480
EP = NS * EP_TILE
N_CHUNKS = EP_TILE // CHUNK
BPB = 4
NB = N_CHUNKS // BPB

BN = 1000
GRID = N // BN


def _seg_kernel(x_hbm, idx_hbm, zeros_hbm, out_hbm,
                buf_a, buf_b, rows_a, rows_b, acc_sh,
                sem_ia, sem_ib, sem_a, sem_b):
    c = lax.axis_index("c")
    s = lax.axis_index("s")

    def fetch_idx(blk, buf, isem):
        pltpu.async_copy(idx_hbm.at[c, s, blk], buf, isem)

    def wait_idx(blk, buf, isem):
        pltpu.make_async_copy(idx_hbm.at[c, s, blk], buf, isem).wait()

    def issue_gather(buf, u, rows, sem):
        pltpu.async_copy(x_hbm.at[buf.at[u, 0]], rows, sem)

    def wait_gather(buf, u, rows, sem):
        pltpu.make_async_copy(x_hbm.at[buf.at[u, 0]], rows, sem).wait()

    def issue_scat(buf, u, rows, sem):
        pltpu.async_copy(rows, acc_sh.at[buf.at[u, 1]], sem, add=True)

    def wait_scat(buf, u, rows, sem):
        pltpu.make_async_copy(rows, acc_sh.at[buf.at[u, 1]], sem).wait()

    fetch_idx(0, buf_a, sem_ia)
    fetch_idx(1, buf_b, sem_ib)

    @pl.loop(0, pl.cdiv(N_ROWCHUNKS, NS))
    def _(i):
        k = i * NS + s

        @pl.when(k < N_ROWCHUNKS)
        def _():
            pltpu.sync_copy(zeros_hbm, acc_sh.at[pl.ds(k * ROWCHUNK, ROWCHUNK)])

    wait_idx(0, buf_a, sem_ia)
    issue_gather(buf_a, 0, rows_a, sem_a)
    issue_gather(buf_a, 1, rows_b, sem_b)
    plsc.subcore_barrier()

    def do_block(blk, buf, isem, obuf, oisem):
        for p in range(BPB // 2):
            u = 2 * p
            wait_gather(buf, u, rows_a, sem_a)
            issue_scat(buf, u, rows_a, sem_a)
            wait_gather(buf, u + 1, rows_b, sem_b)
            issue_scat(buf, u + 1, rows_b, sem_b)
            wait_scat(buf, u, rows_a, sem_a)
            if u + 2 < BPB:
                issue_gather(buf, u + 2, rows_a, sem_a)
            else:
                @pl.when(blk + 1 < NB)
                def _():
                    wait_idx(blk + 1, obuf, oisem)
                    issue_gather(obuf, 0, rows_a, sem_a)
            wait_scat(buf, u + 1, rows_b, sem_b)
            if u + 3 < BPB:
                issue_gather(buf, u + 3, rows_b, sem_b)
            else:
                @pl.when(blk + 1 < NB)
                def _():
                    issue_gather(obuf, 1, rows_b, sem_b)

        @pl.when(blk + 2 < NB)
        def _():
            fetch_idx(blk + 2, buf, isem)

    @pl.loop(0, NB // 2)
    def _(jj):
        do_block(2 * jj, buf_a, sem_ia, buf_b, sem_ib)
        do_block(2 * jj + 1, buf_b, sem_ib, buf_a, sem_ia)

    plsc.subcore_barrier()

    @pl.loop(0, pl.cdiv(N_ROWCHUNKS, NS))
    def _(i):
        k = i * NS + s

        @pl.when(k < N_ROWCHUNKS)
        def _():
            pltpu.sync_copy(acc_sh.at[pl.ds(k * ROWCHUNK, ROWCHUNK)],
                            out_hbm.at[c, pl.ds(k * ROWCHUNK, ROWCHUNK)])


@jax.jit
def _segment_sums(x2flat, idx, zeros):
    mesh = plsc.VectorSubcoreMesh(core_axis_name="c", subcore_axis_name="s",
                                  num_cores=NC, num_subcores=NS)
    kern = pl.kernel(
        _seg_kernel,
        out_type=jax.ShapeDtypeStruct((NC, ACC_ROWS, HH), jnp.float32),
        mesh=mesh,
        scratch_types=[
            pltpu.VMEM((BPB, 2, CHUNK), jnp.int32),
            pltpu.VMEM((BPB, 2, CHUNK), jnp.int32),
            pltpu.VMEM((CHUNK, HH), jnp.float32),
            pltpu.VMEM((CHUNK, HH), jnp.float32),
            pltpu.VMEM_SHARED((ACC_ROWS, HH), jnp.float32),
            pltpu.SemaphoreType.DMA,
            pltpu.SemaphoreType.DMA,
            pltpu.SemaphoreType.DMA,
            pltpu.SemaphoreType.DMA,
        ],
        compiler_params=pltpu.CompilerParams(use_tc_tiling_on_sc=False),
    )
    return kern(x2flat, idx, zeros)


NBH = NB // 2


def _count_kernel(idx_hbm, t_hbm, out_hbm, ibuf, cnt, t_vmem, sem):
    c = lax.axis_index("c")
    s = lax.axis_index("s")
    pltpu.async_copy(t_hbm, t_vmem, sem).wait()
    tcmp = t_vmem[...] + c * N

    @pl.loop(0, ACC_ROWS // 16)
    def _(i):
        cnt[pl.ds(i * 16, 16)] = jnp.zeros((16,), jnp.int32)

    ones = jnp.ones((16,), jnp.int32)

    @pl.loop(0, NBH)
    def _(j):
        pltpu.sync_copy(idx_hbm.at[c, s, c * NBH + j], ibuf)
        for u in range(BPB):
            @pl.loop(0, CHUNK // 16)
            def _(g):
                sv = ibuf[u, 0, pl.ds(g * 16, 16)]
                cv = ibuf[u, 1, pl.ds(g * 16, 16)]
                plsc.addupdate_scatter(cnt, [cv], ones, mask=sv == tcmp)

    pltpu.sync_copy(cnt, out_hbm.at[c, s])


@jax.jit
def _edge_counts(idx, tt):
    mesh = plsc.VectorSubcoreMesh(core_axis_name="c", subcore_axis_name="s",
                                  num_cores=NC, num_subcores=NS)
    kern = pl.kernel(
        _count_kernel,
        out_type=jax.ShapeDtypeStruct((NC, NS, ACC_ROWS), jnp.int32),
        mesh=mesh,
        scratch_types=[
            pltpu.VMEM((BPB, 2, CHUNK), jnp.int32),
            pltpu.VMEM((ACC_ROWS,), jnp.int32),
            pltpu.VMEM((16,), jnp.int32),
            pltpu.SemaphoreType.DMA,
        ],
        compiler_params=pltpu.CompilerParams(use_tc_tiling_on_sc=False,
                                             needs_layout_passes=False),
    )
    return kern(idx, tt)


def _dense_kernel(t_ref, x_ref, a0_ref, a1_ref, a2_ref, rel_ref, W_ref,
                  b_ref, lns_ref, lnb_ref, out_ref):
    j = pl.program_id(0)
    rel = rel_ref[...]

    agg_lo = (a0_ref[0] * rel[0, :HH] + a1_ref[0] * rel[1, :HH]
              + a2_ref[0] * rel[2, :HH])
    agg_hi = (a0_ref[1] * rel[0, HH:] + a1_ref[1] * rel[1, HH:]
              + a2_ref[1] * rel[2, HH:])

    t = t_ref[0]
    row = t - j * BN
    ids = lax.broadcasted_iota(jnp.int32, (BN, 1), 0)
    bmask = (ids == row).astype(jnp.float32)
    agg_lo = agg_lo + bmask
    agg_hi = agg_hi + bmask

    x_lo = x_ref[0]
    x_hi = x_ref[1]
    W = W_ref[...]

    dot = functools.partial(jax.lax.dot_general,
                            dimension_numbers=(((1,), (0,)), ((), ())),
                            preferred_element_type=jnp.float32)
    h = (dot(x_lo, W[0:HH]) + dot(x_hi, W[HH:H])
         + dot(agg_lo, W[H:H + HH]) + dot(agg_hi, W[H + HH:])
         + b_ref[...])

    mean = jnp.mean(h, axis=-1, keepdims=True)
    var = jnp.mean((h - mean) ** 2, axis=-1, keepdims=True)
    h = (h - mean) * lax.rsqrt(var + 1e-5) * lns_ref[...] + lnb_ref[...]
    h = jnp.maximum(h, 0.0)

    out_ref[0] = h[:, :HH] + x_lo
    out_ref[1] = h[:, HH:] + x_hi


@jax.jit
def _dense_layer(t, x2, acc, rel, W, b, lns, lnb):
    a0 = acc[:, 0:N]
    a1 = acc[:, STRIDE:STRIDE + N]
    a2 = acc[:, 2 * STRIDE:2 * STRIDE + N]
    rspec = pl.BlockSpec((NC, BN, HH), lambda j: (0, j, 0))
    return pl.pallas_call(
        _dense_kernel,
        grid=(GRID,),
        in_specs=[
            pl.BlockSpec(memory_space=pltpu.SMEM),
            rspec, rspec, rspec, rspec,
            pl.BlockSpec((R, H), lambda j: (0, 0)),
            pl.BlockSpec((2 * H, H), lambda j: (0, 0)),
            pl.BlockSpec((1, H), lambda j: (0, 0)),
            pl.BlockSpec((1, H), lambda j: (0, 0)),
            pl.BlockSpec((1, H), lambda j: (0, 0)),
        ],
        out_specs=pl.BlockSpec((NC, BN, HH), lambda j: (0, j, 0)),
        out_shape=jax.ShapeDtypeStruct((NC, N, HH), jnp.float32),
        compiler_params=pltpu.CompilerParams(
            allow_input_fusion=[False, False, True, True, True, False,
                                False, False, False, False]),
    )(t, x2, a0, a1, a2, rel, W, b[None], lns[None], lnb[None])


def kernel(edge_index, edge_type, target_token_ids, rel_emb, W, b,
           ln_scale, ln_bias):
    src = edge_index[0]
    dst = edge_index[1]
    t = target_token_ids[0]

    pad = EP - E
    src_p = jnp.concatenate([src, jnp.zeros((pad,), jnp.int32)])
    src2 = jnp.stack([src_p, src_p + N]).reshape(2, NS, NB, BPB, 1, CHUNK)
    cidx = jnp.concatenate([
        edge_type * STRIDE + dst,
        jnp.full((pad,), N, jnp.int32),
    ]).reshape(1, NS, NB, BPB, 1, CHUNK)
    idx = jnp.concatenate(
        [src2, jnp.broadcast_to(cidx, src2.shape)], axis=4)
    zeros = jnp.zeros((ROWCHUNK, HH), jnp.float32)

    x2 = jnp.zeros((NC, N, HH), jnp.float32).at[:, t, :].set(1.0)
    tt = t.reshape(1).astype(jnp.int32)

    for l in range(L):
        if l == 0:
            tt16 = jnp.full((16,), t, jnp.int32)
            cnts = _edge_counts(idx, tt16).sum(axis=(0, 1)).astype(jnp.float32)
            acc = jnp.broadcast_to(cnts[None, :, None], (NC, ACC_ROWS, HH))
        else:
            acc = _segment_sums(x2.reshape(NC * N, HH), idx, zeros)
        x2 = _dense_layer(tt, x2, acc, rel_emb[l], W[l], b[l],
                          ln_scale[l], ln_bias[l])

    return jnp.concatenate([x2[0], x2[1]], axis=-1)[None]
